# baseline (speedup 1.0000x reference)
import functools

import jax
import jax.numpy as jnp
import numpy as np
from jax import lax
from jax.experimental import pallas as pl
from jax.experimental.pallas import tpu as pltpu
from jax.experimental.pallas import tpu_sc as plsc

_DIM = 64
_NW = 32
_TOK = 128
_NBUF = 4

_PERM = np.stack([np.arange(64), np.arange(64) + 64], axis=1).reshape(-1)


@functools.lru_cache(maxsize=None)
def _make_gather(n_rows: int, n_cols: int):
    mesh = plsc.VectorSubcoreMesh(core_axis_name="c", subcore_axis_name="s")

    @functools.partial(
        pl.kernel,
        out_type=jax.ShapeDtypeStruct((n_rows * n_cols, _DIM), jnp.float32),
        mesh=mesh,
        compiler_params=pltpu.CompilerParams(use_tc_tiling_on_sc=False),
        scratch_types=[pltpu.VMEM((n_cols, _TOK), jnp.int32)]
        + [pltpu.VMEM((_TOK, _DIM), jnp.float32)] * _NBUF
        + [pltpu.SemaphoreType.DMA] * (2 * _NBUF),
    )
    def k(idx_hbm, table_hbm, out_hbm, idx_v, *rest):
        rows = rest[:_NBUF]
        gsem = rest[_NBUF:2 * _NBUF]
        osem = rest[2 * _NBUF:]
        wid = lax.axis_index("s") * 2 + lax.axis_index("c")
        i0 = wid * _TOK
        pltpu.sync_copy(idx_hbm.at[wid], idx_v)

        def fire_gather(b, j):
            pltpu.async_copy(table_hbm.at[idx_v.at[j]], rows[b], gsem[b])

        def wait_gather(b, j):
            pltpu.make_async_copy(table_hbm.at[idx_v.at[j]], rows[b], gsem[b]).wait()

        def out_slice(j):
            return out_hbm.at[pl.ds(j * n_rows + i0, _TOK)]

        def fire_out(b, j):
            pltpu.async_copy(rows[b], out_slice(j), osem[b])

        def wait_out(b, j):
            pltpu.make_async_copy(rows[b], out_slice(j), osem[b]).wait()

        n_grp = n_cols // _NBUF

        for b in range(_NBUF):
            fire_gather(b, b)

        @pl.loop(0, n_grp - 1)
        def _(g):
            j0 = g * _NBUF
            for b in range(_NBUF):
                wait_gather(b, j0 + b)
                fire_out(b, j0 + b)
            for b in range(_NBUF):
                wait_out(b, j0 + b)
                fire_gather(b, j0 + _NBUF + b)

        j0 = (n_grp - 1) * _NBUF
        for b in range(_NBUF):
            wait_gather(b, j0 + b)
            fire_out(b, j0 + b)
        for b in range(_NBUF):
            wait_out(b, j0 + b)

    return k


def _tc_body(x_ref, o_ref):
    x = x_ref[...]
    a = x[:, :_DIM]
    b = x[:, _DIM:]
    o_ref[0] = jnp.concatenate([a.T, b.T], axis=1)


@functools.lru_cache(maxsize=None)
def _make_transpose(n_rows: int, n_cols: int):
    nblk = n_rows // _TOK
    return pl.pallas_call(
        _tc_body,
        grid=(n_cols, nblk),
        in_specs=[
            pl.BlockSpec((_DIM, 2 * _DIM), lambda j, ti: (j * nblk + ti, 0)),
        ],
        out_specs=pl.BlockSpec((1, _DIM, _TOK), lambda j, ti: (j, 0, ti)),
        out_shape=jax.ShapeDtypeStruct((n_cols, _DIM, n_rows), jnp.float32),
    )


def kernel(emb_indices, table):
    n_rows, n_cols = emb_indices.shape
    idx_w = emb_indices.T.reshape(n_cols, _NW, _TOK).transpose(1, 0, 2)[:, :, _PERM]
    r = _make_gather(n_rows, n_cols)(idx_w, table)
    r128 = r.reshape(n_rows * n_cols // 2, 2 * _DIM)
    p = _make_transpose(n_rows, n_cols)(r128)
    return jnp.transpose(p, (2, 0, 1))


# speedup vs baseline: 5.3893x; 5.3893x over previous
import functools

import jax
import jax.numpy as jnp
from jax import lax
from jax.experimental import pallas as pl
from jax.experimental.pallas import tpu as pltpu
from jax.experimental.pallas import tpu_sc as plsc

_DIM = 64
_NW = 32
_CHUNK = 128
_NBUF = 8


@functools.lru_cache(maxsize=None)
def _make_gather(n_flat: int):
    mesh = plsc.VectorSubcoreMesh(core_axis_name="c", subcore_axis_name="s")
    per_w = n_flat // _NW
    n_chunk = per_w // _CHUNK
    n_grp = n_chunk // _NBUF

    @functools.partial(
        pl.kernel,
        out_type=jax.ShapeDtypeStruct((n_flat, _DIM), jnp.float32),
        mesh=mesh,
        compiler_params=pltpu.CompilerParams(use_tc_tiling_on_sc=False),
        scratch_types=[pltpu.VMEM((n_chunk, _CHUNK), jnp.int32)]
        + [pltpu.VMEM((_CHUNK, _DIM), jnp.float32)] * _NBUF
        + [pltpu.SemaphoreType.DMA] * (2 * _NBUF),
    )
    def k(idx_hbm, table_hbm, out_hbm, idx_v, *rest):
        rows = rest[:_NBUF]
        gsem = rest[_NBUF:2 * _NBUF]
        osem = rest[2 * _NBUF:]
        wid = lax.axis_index("s") * 2 + lax.axis_index("c")
        base = wid * per_w
        pltpu.sync_copy(idx_hbm.at[wid], idx_v)

        def fire_gather(b, c):
            pltpu.async_copy(table_hbm.at[idx_v.at[c]], rows[b], gsem[b])

        def wait_gather(b, c):
            pltpu.make_async_copy(table_hbm.at[idx_v.at[c]], rows[b], gsem[b]).wait()

        def out_slice(c):
            return out_hbm.at[pl.ds(base + c * _CHUNK, _CHUNK)]

        def fire_out(b, c):
            pltpu.async_copy(rows[b], out_slice(c), osem[b])

        def wait_out(b, c):
            pltpu.make_async_copy(rows[b], out_slice(c), osem[b]).wait()

        for b in range(_NBUF):
            fire_gather(b, b)

        @pl.loop(0, n_grp - 1)
        def _(g):
            c0 = g * _NBUF
            for b in range(_NBUF):
                wait_gather(b, c0 + b)
                fire_out(b, c0 + b)
            for b in range(_NBUF):
                wait_out(b, c0 + b)
                fire_gather(b, c0 + _NBUF + b)

        c0 = (n_grp - 1) * _NBUF
        for b in range(_NBUF):
            wait_gather(b, c0 + b)
            fire_out(b, c0 + b)
        for b in range(_NBUF):
            wait_out(b, c0 + b)

    return k


def kernel(emb_indices, table):
    n_rows, n_cols = emb_indices.shape
    idx_w = emb_indices.reshape(_NW, -1, _CHUNK)
    r = _make_gather(n_rows * n_cols)(idx_w, table)
    return r.reshape(n_rows, n_cols, _DIM)


# speedup vs baseline: 5.3973x; 1.0015x over previous
import functools

import jax
import jax.numpy as jnp
from jax import lax
from jax.experimental import pallas as pl
from jax.experimental.pallas import tpu as pltpu
from jax.experimental.pallas import tpu_sc as plsc

_DIM = 64
_NW = 32
_CHUNK = 128
_NBUF = 10


@functools.lru_cache(maxsize=None)
def _make_gather(n_flat: int):
    mesh = plsc.VectorSubcoreMesh(core_axis_name="c", subcore_axis_name="s")
    per_w = n_flat // _NW
    n_chunk = per_w // _CHUNK
    n_grp = n_chunk // _NBUF

    @functools.partial(
        pl.kernel,
        out_type=jax.ShapeDtypeStruct((n_flat, _DIM), jnp.float32),
        mesh=mesh,
        compiler_params=pltpu.CompilerParams(use_tc_tiling_on_sc=False),
        scratch_types=[pltpu.VMEM((n_chunk, _CHUNK), jnp.int32)]
        + [pltpu.VMEM((_CHUNK, _DIM), jnp.float32)] * _NBUF
        + [pltpu.SemaphoreType.DMA] * (2 * _NBUF),
    )
    def k(idx_hbm, table_hbm, out_hbm, idx_v, *rest):
        rows = rest[:_NBUF]
        gsem = rest[_NBUF:2 * _NBUF]
        osem = rest[2 * _NBUF:]
        wid = lax.axis_index("s") * 2 + lax.axis_index("c")
        base = wid * per_w
        pltpu.sync_copy(idx_hbm.at[wid], idx_v)

        def fire_gather(b, c):
            pltpu.async_copy(table_hbm.at[idx_v.at[c]], rows[b], gsem[b])

        def wait_gather(b, c):
            pltpu.make_async_copy(table_hbm.at[idx_v.at[c]], rows[b], gsem[b]).wait()

        def out_slice(c):
            return out_hbm.at[pl.ds(base + c * _CHUNK, _CHUNK)]

        def fire_out(b, c):
            pltpu.async_copy(rows[b], out_slice(c), osem[b])

        def wait_out(b, c):
            pltpu.make_async_copy(rows[b], out_slice(c), osem[b]).wait()

        for b in range(_NBUF):
            fire_gather(b, b)

        @pl.loop(0, n_grp - 1)
        def _(g):
            c0 = g * _NBUF
            for b in range(_NBUF):
                wait_gather(b, c0 + b)
                fire_out(b, c0 + b)
            for b in range(_NBUF):
                wait_out(b, c0 + b)
                fire_gather(b, c0 + _NBUF + b)

        c0 = (n_grp - 1) * _NBUF
        for b in range(_NBUF):
            wait_gather(b, c0 + b)
            fire_out(b, c0 + b)
        for b in range(_NBUF):
            wait_out(b, c0 + b)

    return k


def kernel(emb_indices, table):
    n_rows, n_cols = emb_indices.shape
    idx_w = emb_indices.reshape(_NW, -1, _CHUNK)
    r = _make_gather(n_rows * n_cols)(idx_w, table)
    return r.reshape(n_rows, n_cols, _DIM)


# speedup vs baseline: 5.4086x; 1.0021x over previous
import functools

import jax
import jax.numpy as jnp
from jax import lax
from jax.experimental import pallas as pl
from jax.experimental.pallas import tpu as pltpu
from jax.experimental.pallas import tpu_sc as plsc

_DIM = 64
_NW = 32
_CHUNK = 128
_NSLOT = 5
_PAIR = 2 * _CHUNK


@functools.lru_cache(maxsize=None)
def _make_gather(n_flat: int):
    mesh = plsc.VectorSubcoreMesh(core_axis_name="c", subcore_axis_name="s")
    per_w = n_flat // _NW
    n_chunk = per_w // _CHUNK
    n_pair = n_chunk // 2
    n_grp = n_pair // _NSLOT

    @functools.partial(
        pl.kernel,
        out_type=jax.ShapeDtypeStruct((n_flat, _DIM), jnp.float32),
        mesh=mesh,
        compiler_params=pltpu.CompilerParams(use_tc_tiling_on_sc=False),
        scratch_types=[
            pltpu.VMEM((n_chunk, _CHUNK), jnp.int32),
            pltpu.VMEM((_NSLOT * _PAIR, _DIM), jnp.float32),
        ]
        + [pltpu.SemaphoreType.DMA] * (3 * _NSLOT),
    )
    def k(idx_hbm, table_hbm, out_hbm, idx_v, buf, *sems):
        gsem = sems[:2 * _NSLOT]
        osem = sems[2 * _NSLOT:]
        wid = lax.axis_index("s") * 2 + lax.axis_index("c")
        base = wid * per_w
        pltpu.sync_copy(idx_hbm.at[wid], idx_v)

        def g_dst(s, o):
            return buf.at[pl.ds(s * _PAIR + o * _CHUNK, _CHUNK)]

        def fire_gather(s, o, p):
            pltpu.async_copy(
                table_hbm.at[idx_v.at[2 * p + o]], g_dst(s, o), gsem[2 * s + o])

        def wait_gather(s, o, p):
            pltpu.make_async_copy(
                table_hbm.at[idx_v.at[2 * p + o]], g_dst(s, o), gsem[2 * s + o]).wait()

        def o_src(s):
            return buf.at[pl.ds(s * _PAIR, _PAIR)]

        def o_dst(p):
            return out_hbm.at[pl.ds(base + p * _PAIR, _PAIR)]

        def fire_out(s, p):
            pltpu.async_copy(o_src(s), o_dst(p), osem[s])

        def wait_out(s, p):
            pltpu.make_async_copy(o_src(s), o_dst(p), osem[s]).wait()

        for s in range(_NSLOT):
            fire_gather(s, 0, s)
            fire_gather(s, 1, s)

        @pl.loop(0, n_grp - 1)
        def _(g):
            p0 = g * _NSLOT
            for s in range(_NSLOT):
                wait_gather(s, 0, p0 + s)
                wait_gather(s, 1, p0 + s)
                fire_out(s, p0 + s)
            for s in range(_NSLOT):
                wait_out(s, p0 + s)
                fire_gather(s, 0, p0 + _NSLOT + s)
                fire_gather(s, 1, p0 + _NSLOT + s)

        p0 = (n_grp - 1) * _NSLOT
        for s in range(_NSLOT):
            wait_gather(s, 0, p0 + s)
            wait_gather(s, 1, p0 + s)
            fire_out(s, p0 + s)
        for s in range(_NSLOT):
            wait_out(s, p0 + s)

    return k


def kernel(emb_indices, table):
    n_rows, n_cols = emb_indices.shape
    idx_w = emb_indices.reshape(_NW, -1, _CHUNK)
    r = _make_gather(n_rows * n_cols)(idx_w, table)
    return r.reshape(n_rows, n_cols, _DIM)
